# baseline (speedup 1.0000x reference)
import functools

import jax
import jax.numpy as jnp
from jax import lax
from jax.experimental import pallas as pl
from jax.experimental.pallas import tpu as pltpu
from jax.experimental.pallas import tpu_sc as plsc

N = 10000
E = 320000
D = 128
NC = 2
NS = 16
NW = NC * NS
G = 128
GPW = 80
NGROUPS = NW * GPW
EPAD = NGROUPS * G
CH = 4
NCHUNK = GPW // CH

ROWS_PER_TILE = N // NS

_mesh = plsc.VectorSubcoreMesh(core_axis_name="c", subcore_axis_name="s")
_sc_params = pltpu.CompilerParams(needs_layout_passes=False,
                                  use_tc_tiling_on_sc=False)


@functools.partial(
    pl.kernel,
    out_type=jax.ShapeDtypeStruct((NW, N), jnp.float32),
    mesh=_mesh,
    scratch_types=[
        pltpu.VMEM((CH, G), jnp.int32),
        pltpu.VMEM((CH, G), jnp.float32),
        pltpu.VMEM((N,), jnp.float32),
    ],
    compiler_params=_sc_params,
)
def _deg_kernel(col_hbm, ew_hbm, degp_hbm, colbuf, ewbuf, deg_local):
    c = lax.axis_index("c")
    s = lax.axis_index("s")
    wid = s * NC + c

    zero16 = jnp.zeros((16,), jnp.float32)

    def zbody(i, carry):
        deg_local[pl.ds(i * 16, 16)] = zero16
        return carry

    lax.fori_loop(0, N // 16, zbody, 0)

    lo = wid * GPW

    def cbody(ci, carry):
        gb = lo + ci * CH
        pltpu.sync_copy(col_hbm.at[pl.ds(gb, CH)], colbuf)
        pltpu.sync_copy(ew_hbm.at[pl.ds(gb, CH)], ewbuf)

        def inner(k, carry2):
            o = k // (G // 16)
            e0 = (k % (G // 16)) * 16
            cv = colbuf[o, pl.ds(e0, 16)]
            wv = ewbuf[o, pl.ds(e0, 16)]
            plsc.addupdate_scatter(deg_local, [cv], wv)
            return carry2

        lax.fori_loop(0, CH * (G // 16), inner, 0)
        return carry

    lax.fori_loop(0, NCHUNK, cbody, 0)
    pltpu.sync_copy(deg_local, degp_hbm.at[wid])


@functools.partial(
    pl.kernel,
    out_type=jax.ShapeDtypeStruct((NC, N, D), jnp.float32),
    mesh=_mesh,
    scratch_types=[
        pltpu.VMEM((CH, G), jnp.int32),
        pltpu.VMEM((CH, G), jnp.int32),
        pltpu.VMEM((CH, G), jnp.float32),
        pltpu.VMEM((2, G, D), jnp.float32),
        pltpu.VMEM_SHARED((N, D), jnp.float32),
        pltpu.SemaphoreType.DMA,
        pltpu.SemaphoreType.DMA,
        pltpu.SemaphoreType.DMA,
        pltpu.SemaphoreType.DMA,
    ],
    compiler_params=_sc_params,
)
def _layer_kernel(row_hbm, col_hbm, ew_hbm, h_hbm, zeros_hbm, outp_hbm,
                  rowidx, colidx, ewbuf, rows, acc,
                  gsem0, gsem1, ssem0, ssem1):
    c = lax.axis_index("c")
    s = lax.axis_index("s")
    wid = s * NC + c
    gsems = (gsem0, gsem1)
    ssems = (ssem0, ssem1)

    pltpu.sync_copy(zeros_hbm.at[pl.ds(s * ROWS_PER_TILE, ROWS_PER_TILE)],
                    acc.at[pl.ds(s * ROWS_PER_TILE, ROWS_PER_TILE)])
    plsc.subcore_barrier()

    lo = wid * GPW

    def fire_gather(o, b):
        pltpu.async_copy(h_hbm.at[rowidx.at[o]], rows.at[b], gsems[b])

    def wait_gather(o, b):
        pltpu.make_async_copy(h_hbm.at[rowidx.at[o]], rows.at[b],
                              gsems[b]).wait()

    def fire_scatter(o, b):
        pltpu.async_copy(rows.at[b], acc.at[colidx.at[o]], ssems[b], add=True)

    def wait_scatter(o, b):
        pltpu.make_async_copy(rows.at[b], acc.at[colidx.at[o]],
                              ssems[b]).wait()

    def scale(o, b):
        def sbody(k, carry2):
            e0 = k * 16
            for j in range(16):
                w16 = plsc.load_gather(
                    ewbuf.at[o], [jnp.full((16,), e0 + j, jnp.int32)])
                for f in range(D // 16):
                    sl = pl.ds(f * 16, 16)
                    rows[b, e0 + j, sl] = rows[b, e0 + j, sl] * w16
            return carry2

        lax.fori_loop(0, G // 16, sbody, 0)

    def cbody(ci, carry):
        gb = lo + ci * CH
        pltpu.sync_copy(row_hbm.at[pl.ds(gb, CH)], rowidx)
        pltpu.sync_copy(col_hbm.at[pl.ds(gb, CH)], colidx)
        pltpu.sync_copy(ew_hbm.at[pl.ds(gb, CH)], ewbuf)
        fire_gather(0, 0)
        for o in range(CH):
            b = o % 2
            nb = 1 - b
            if o + 1 < CH:
                if o >= 1:
                    wait_scatter(o - 1, nb)
                fire_gather(o + 1, nb)
            wait_gather(o, b)
            scale(o, b)
            fire_scatter(o, b)
        wait_scatter(CH - 2, 0 if (CH - 2) % 2 == 0 else 1)
        wait_scatter(CH - 1, 0 if (CH - 1) % 2 == 0 else 1)
        return carry

    lax.fori_loop(0, NCHUNK, cbody, 0)
    plsc.subcore_barrier()

    pltpu.sync_copy(acc.at[pl.ds(s * ROWS_PER_TILE, ROWS_PER_TILE)],
                    outp_hbm.at[c, pl.ds(s * ROWS_PER_TILE, ROWS_PER_TILE)])


def _dis_body(degp_ref, dis_ref):
    deg = jnp.sum(degp_ref[...], axis=0, keepdims=True) + 1.0
    safe = jnp.where(deg > 0, deg, 1.0)
    dis_ref[...] = jnp.where(deg > 0, lax.rsqrt(safe), 0.0)


def _dis_call(degp):
    return pl.pallas_call(
        _dis_body,
        out_shape=jax.ShapeDtypeStruct((1, N), jnp.float32),
    )(degp)


_BLK = 2000
_NBLK = N // _BLK


def _mm_body(x_ref, w_ref, disc_ref, out_ref):
    h = lax.dot_general(x_ref[...], w_ref[...], (((1,), (1,)), ((), ())),
                        preferred_element_type=jnp.float32)
    out_ref[...] = h * disc_ref[...]


def _mm_call(x, w, disc):
    return pl.pallas_call(
        _mm_body,
        grid=(_NBLK,),
        in_specs=[
            pl.BlockSpec((_BLK, D), lambda i: (i, 0)),
            pl.BlockSpec((D, D), lambda i: (0, 0)),
            pl.BlockSpec((_BLK, 1), lambda i: (i, 0)),
        ],
        out_specs=pl.BlockSpec((_BLK, D), lambda i: (i, 0)),
        out_shape=jax.ShapeDtypeStruct((N, D), jnp.float32),
    )(x, w, disc)


def _mid_body(p_ref, hp_ref, disc_ref, b_ref, w_ref, out_ref):
    srow = p_ref[0] + p_ref[1] + hp_ref[...]
    z = jnp.maximum(disc_ref[...] * srow + b_ref[...], 0.0)
    h2 = lax.dot_general(z, w_ref[...], (((1,), (1,)), ((), ())),
                         preferred_element_type=jnp.float32)
    out_ref[...] = h2 * disc_ref[...]


def _mid_call(p, hp, disc, b, w):
    return pl.pallas_call(
        _mid_body,
        grid=(_NBLK,),
        in_specs=[
            pl.BlockSpec((NC, _BLK, D), lambda i: (0, i, 0)),
            pl.BlockSpec((_BLK, D), lambda i: (i, 0)),
            pl.BlockSpec((_BLK, 1), lambda i: (i, 0)),
            pl.BlockSpec((1, D), lambda i: (0, 0)),
            pl.BlockSpec((D, D), lambda i: (0, 0)),
        ],
        out_specs=pl.BlockSpec((_BLK, D), lambda i: (i, 0)),
        out_shape=jax.ShapeDtypeStruct((N, D), jnp.float32),
    )(p, hp, disc, b, w)


def _final_body(p_ref, hp_ref, disc_ref, b_ref, out_ref):
    srow = p_ref[0] + p_ref[1] + hp_ref[...]
    out_ref[...] = jnp.maximum(disc_ref[...] * srow + b_ref[...], 0.0)


def _final_call(p, hp, disc, b):
    return pl.pallas_call(
        _final_body,
        grid=(_NBLK,),
        in_specs=[
            pl.BlockSpec((NC, _BLK, D), lambda i: (0, i, 0)),
            pl.BlockSpec((_BLK, D), lambda i: (i, 0)),
            pl.BlockSpec((_BLK, 1), lambda i: (i, 0)),
            pl.BlockSpec((1, D), lambda i: (0, 0)),
        ],
        out_specs=pl.BlockSpec((_BLK, D), lambda i: (i, 0)),
        out_shape=jax.ShapeDtypeStruct((N, D), jnp.float32),
    )(p, hp, disc, b)


def kernel(x, edge_index, edge_weights, W1, b1, W2, b2):
    npad = EPAD - E
    row = jnp.concatenate(
        [edge_index[0], jnp.zeros((npad,), jnp.int32)]).reshape(NGROUPS, G)
    col = jnp.concatenate(
        [edge_index[1], jnp.zeros((npad,), jnp.int32)]).reshape(NGROUPS, G)
    ew = jnp.concatenate(
        [edge_weights, jnp.zeros((npad,), jnp.float32)]).reshape(NGROUPS, G)
    zeros_nd = jnp.zeros((N, D), jnp.float32)

    degp = _deg_kernel(col, ew)
    dis = _dis_call(degp)
    disc = dis.reshape(N, 1)

    h1p = _mm_call(x, W1, disc)
    p1 = _layer_kernel(row, col, ew, h1p, zeros_nd)
    h2p = _mid_call(p1, h1p, disc, b1.reshape(1, D), W2)
    p2 = _layer_kernel(row, col, ew, h2p, zeros_nd)
    return _final_call(p2, h2p, disc, b2.reshape(1, D))


# speedup vs baseline: 2.4220x; 2.4220x over previous
import functools

import jax
import jax.numpy as jnp
from jax import lax
from jax.experimental import pallas as pl
from jax.experimental.pallas import tpu as pltpu
from jax.experimental.pallas import tpu_sc as plsc

N = 10000
E = 320000
D = 128
NC = 2
NS = 16
NW = NC * NS
G = 128
GPW = 80
NGROUPS = NW * GPW
EPAD = NGROUPS * G
CH = 8
NCHUNK = GPW // CH

ROWS_PER_TILE = N // NS

_mesh = plsc.VectorSubcoreMesh(core_axis_name="c", subcore_axis_name="s")
_sc_params = pltpu.CompilerParams(needs_layout_passes=False,
                                  use_tc_tiling_on_sc=False)


@functools.partial(
    pl.kernel,
    out_type=jax.ShapeDtypeStruct((NW, N), jnp.float32),
    mesh=_mesh,
    scratch_types=[
        pltpu.VMEM((CH, G), jnp.int32),
        pltpu.VMEM((CH, G), jnp.float32),
        pltpu.VMEM((N,), jnp.float32),
    ],
    compiler_params=_sc_params,
)
def _deg_kernel(col_hbm, ew_hbm, degp_hbm, colbuf, ewbuf, deg_local):
    c = lax.axis_index("c")
    s = lax.axis_index("s")
    wid = s * NC + c

    zero16 = jnp.zeros((16,), jnp.float32)

    def zbody(i, carry):
        deg_local[pl.ds(i * 16, 16)] = zero16
        return carry

    lax.fori_loop(0, N // 16, zbody, 0)

    lo = wid * GPW

    def cbody(ci, carry):
        gb = lo + ci * CH
        pltpu.sync_copy(col_hbm.at[pl.ds(gb, CH)], colbuf)
        pltpu.sync_copy(ew_hbm.at[pl.ds(gb, CH)], ewbuf)

        def inner(k, carry2):
            o = k // (G // 16)
            e0 = (k % (G // 16)) * 16
            cv = colbuf[o, pl.ds(e0, 16)]
            wv = ewbuf[o, pl.ds(e0, 16)]
            plsc.addupdate_scatter(deg_local, [cv], wv)
            return carry2

        lax.fori_loop(0, CH * (G // 16), inner, 0)
        return carry

    lax.fori_loop(0, NCHUNK, cbody, 0)
    pltpu.sync_copy(deg_local, degp_hbm.at[wid])


@functools.partial(
    pl.kernel,
    out_type=jax.ShapeDtypeStruct((NC, N, D), jnp.float32),
    mesh=_mesh,
    scratch_types=[
        pltpu.VMEM((CH, G), jnp.int32),
        pltpu.VMEM((CH, G), jnp.int32),
        pltpu.VMEM((CH, G), jnp.float32),
        pltpu.VMEM((2, G, D), jnp.float32),
        pltpu.VMEM_SHARED((N, D), jnp.float32),
        pltpu.SemaphoreType.DMA,
        pltpu.SemaphoreType.DMA,
        pltpu.SemaphoreType.DMA,
        pltpu.SemaphoreType.DMA,
    ],
    compiler_params=_sc_params,
)
def _layer_kernel(row_hbm, col_hbm, ew_hbm, h_hbm, zeros_hbm, outp_hbm,
                  rowidx, colidx, ewbuf, rows, acc,
                  gsem0, gsem1, ssem0, ssem1):
    c = lax.axis_index("c")
    s = lax.axis_index("s")
    wid = s * NC + c
    gsems = (gsem0, gsem1)
    ssems = (ssem0, ssem1)

    pltpu.sync_copy(zeros_hbm.at[pl.ds(s * ROWS_PER_TILE, ROWS_PER_TILE)],
                    acc.at[pl.ds(s * ROWS_PER_TILE, ROWS_PER_TILE)])
    plsc.subcore_barrier()

    lo = wid * GPW

    def fire_gather(o, b):
        pltpu.async_copy(h_hbm.at[rowidx.at[o]], rows.at[b], gsems[b])

    def wait_gather(o, b):
        pltpu.make_async_copy(h_hbm.at[rowidx.at[o]], rows.at[b],
                              gsems[b]).wait()

    def fire_scatter(o, b):
        pltpu.async_copy(rows.at[b], acc.at[colidx.at[o]], ssems[b], add=True)

    def wait_scatter(o, b):
        pltpu.make_async_copy(rows.at[b], acc.at[colidx.at[o]],
                              ssems[b]).wait()

    def scale(o, b):
        def sbody(k, carry2):
            e0 = k * 16
            for j in range(16):
                w16 = plsc.load_gather(
                    ewbuf.at[o], [jnp.full((16,), e0 + j, jnp.int32)])
                for f in range(D // 16):
                    sl = pl.ds(f * 16, 16)
                    rows[b, e0 + j, sl] = rows[b, e0 + j, sl] * w16
            return carry2

        lax.fori_loop(0, G // 16, sbody, 0)

    def cbody(ci, carry):
        gb = lo + ci * CH
        pltpu.sync_copy(row_hbm.at[pl.ds(gb, CH)], rowidx)
        pltpu.sync_copy(col_hbm.at[pl.ds(gb, CH)], colidx)
        pltpu.sync_copy(ew_hbm.at[pl.ds(gb, CH)], ewbuf)
        fire_gather(0, 0)
        for o in range(CH):
            b = o % 2
            nb = 1 - b
            if o + 1 < CH:
                if o >= 1:
                    wait_scatter(o - 1, nb)
                fire_gather(o + 1, nb)
            wait_gather(o, b)
            scale(o, b)
            fire_scatter(o, b)
        wait_scatter(CH - 2, 0 if (CH - 2) % 2 == 0 else 1)
        wait_scatter(CH - 1, 0 if (CH - 1) % 2 == 0 else 1)
        return carry

    lax.fori_loop(0, NCHUNK, cbody, 0)
    plsc.subcore_barrier()

    pltpu.sync_copy(acc.at[pl.ds(s * ROWS_PER_TILE, ROWS_PER_TILE)],
                    outp_hbm.at[c, pl.ds(s * ROWS_PER_TILE, ROWS_PER_TILE)])


def _dis_body(degp_ref, dis_ref):
    deg = jnp.sum(degp_ref[...], axis=0, keepdims=True) + 1.0
    safe = jnp.where(deg > 0, deg, 1.0)
    dis_ref[...] = jnp.where(deg > 0, lax.rsqrt(safe), 0.0)


def _dis_call(degp):
    return pl.pallas_call(
        _dis_body,
        out_shape=jax.ShapeDtypeStruct((1, N), jnp.float32),
    )(degp)


_BLK = 2000
_NBLK = N // _BLK


def _mm_body(x_ref, w_ref, disc_ref, out_ref):
    h = lax.dot_general(x_ref[...], w_ref[...], (((1,), (1,)), ((), ())),
                        preferred_element_type=jnp.float32)
    out_ref[...] = h * disc_ref[...]


def _mm_call(x, w, disc):
    return pl.pallas_call(
        _mm_body,
        grid=(_NBLK,),
        in_specs=[
            pl.BlockSpec((_BLK, D), lambda i: (i, 0)),
            pl.BlockSpec((D, D), lambda i: (0, 0)),
            pl.BlockSpec((_BLK, 1), lambda i: (i, 0)),
        ],
        out_specs=pl.BlockSpec((_BLK, D), lambda i: (i, 0)),
        out_shape=jax.ShapeDtypeStruct((N, D), jnp.float32),
    )(x, w, disc)


def _mid_body(p_ref, hp_ref, disc_ref, b_ref, w_ref, out_ref):
    srow = p_ref[0] + p_ref[1] + hp_ref[...]
    z = jnp.maximum(disc_ref[...] * srow + b_ref[...], 0.0)
    h2 = lax.dot_general(z, w_ref[...], (((1,), (1,)), ((), ())),
                         preferred_element_type=jnp.float32)
    out_ref[...] = h2 * disc_ref[...]


def _mid_call(p, hp, disc, b, w):
    return pl.pallas_call(
        _mid_body,
        grid=(_NBLK,),
        in_specs=[
            pl.BlockSpec((NC, _BLK, D), lambda i: (0, i, 0)),
            pl.BlockSpec((_BLK, D), lambda i: (i, 0)),
            pl.BlockSpec((_BLK, 1), lambda i: (i, 0)),
            pl.BlockSpec((1, D), lambda i: (0, 0)),
            pl.BlockSpec((D, D), lambda i: (0, 0)),
        ],
        out_specs=pl.BlockSpec((_BLK, D), lambda i: (i, 0)),
        out_shape=jax.ShapeDtypeStruct((N, D), jnp.float32),
    )(p, hp, disc, b, w)


def _final_body(p_ref, hp_ref, disc_ref, b_ref, out_ref):
    srow = p_ref[0] + p_ref[1] + hp_ref[...]
    out_ref[...] = jnp.maximum(disc_ref[...] * srow + b_ref[...], 0.0)


def _final_call(p, hp, disc, b):
    return pl.pallas_call(
        _final_body,
        grid=(_NBLK,),
        in_specs=[
            pl.BlockSpec((NC, _BLK, D), lambda i: (0, i, 0)),
            pl.BlockSpec((_BLK, D), lambda i: (i, 0)),
            pl.BlockSpec((_BLK, 1), lambda i: (i, 0)),
            pl.BlockSpec((1, D), lambda i: (0, 0)),
        ],
        out_specs=pl.BlockSpec((_BLK, D), lambda i: (i, 0)),
        out_shape=jax.ShapeDtypeStruct((N, D), jnp.float32),
    )(p, hp, disc, b)


def kernel(x, edge_index, edge_weights, W1, b1, W2, b2):
    npad = EPAD - E
    spread = (jnp.arange(npad, dtype=jnp.int32) * 16) % N
    row = jnp.concatenate(
        [edge_index[0], spread]).reshape(NGROUPS, G)
    col = jnp.concatenate(
        [edge_index[1], spread]).reshape(NGROUPS, G)
    ew = jnp.concatenate(
        [edge_weights, jnp.zeros((npad,), jnp.float32)]).reshape(NGROUPS, G)
    zeros_nd = jnp.zeros((N, D), jnp.float32)

    degp = _deg_kernel(col, ew)
    dis = _dis_call(degp)
    disc = dis.reshape(N, 1)

    h1p = _mm_call(x, W1, disc)
    p1 = _layer_kernel(row, col, ew, h1p, zeros_nd)
    h2p = _mid_call(p1, h1p, disc, b1.reshape(1, D), W2)
    p2 = _layer_kernel(row, col, ew, h2p, zeros_nd)
    return _final_call(p2, h2p, disc, b2.reshape(1, D))


# speedup vs baseline: 2.5779x; 1.0644x over previous
import functools

import jax
import jax.numpy as jnp
from jax import lax
from jax.experimental import pallas as pl
from jax.experimental.pallas import tpu as pltpu
from jax.experimental.pallas import tpu_sc as plsc

N = 10000
E = 320000
D = 128
NC = 2
NS = 16
NW = NC * NS
G = 128
GPW = 80
NGROUPS = NW * GPW
EPAD = NGROUPS * G
CH = 8
NCHUNK = GPW // CH

ROWS_PER_TILE = N // NS

_mesh = plsc.VectorSubcoreMesh(core_axis_name="c", subcore_axis_name="s")
_sc_params = pltpu.CompilerParams(needs_layout_passes=False,
                                  use_tc_tiling_on_sc=False)


@functools.partial(
    pl.kernel,
    out_type=jax.ShapeDtypeStruct((NW, N), jnp.float32),
    mesh=_mesh,
    scratch_types=[
        pltpu.VMEM((CH, G), jnp.int32),
        pltpu.VMEM((CH, G), jnp.float32),
        pltpu.VMEM((N,), jnp.float32),
    ],
    compiler_params=_sc_params,
)
def _deg_kernel(col_hbm, ew_hbm, degp_hbm, colbuf, ewbuf, deg_local):
    c = lax.axis_index("c")
    s = lax.axis_index("s")
    wid = s * NC + c

    zero16 = jnp.zeros((16,), jnp.float32)

    def zbody(i, carry):
        deg_local[pl.ds(i * 16, 16)] = zero16
        return carry

    lax.fori_loop(0, N // 16, zbody, 0)

    lo = wid * GPW

    def cbody(ci, carry):
        gb = lo + ci * CH
        pltpu.sync_copy(col_hbm.at[pl.ds(gb, CH)], colbuf)
        pltpu.sync_copy(ew_hbm.at[pl.ds(gb, CH)], ewbuf)

        def inner(k, carry2):
            o = k // (G // 16)
            e0 = (k % (G // 16)) * 16
            cv = colbuf[o, pl.ds(e0, 16)]
            wv = ewbuf[o, pl.ds(e0, 16)]
            plsc.addupdate_scatter(deg_local, [cv], wv)
            return carry2

        lax.fori_loop(0, CH * (G // 16), inner, 0)
        return carry

    lax.fori_loop(0, NCHUNK, cbody, 0)
    pltpu.sync_copy(deg_local, degp_hbm.at[wid])


@functools.partial(
    pl.kernel,
    out_type=jax.ShapeDtypeStruct((NC, N, D), jnp.float32),
    mesh=_mesh,
    scratch_types=[
        pltpu.VMEM((2, CH, G), jnp.int32),
        pltpu.VMEM((2, CH, G), jnp.int32),
        pltpu.VMEM((2, CH, G), jnp.float32),
        pltpu.VMEM((2, G, D), jnp.float32),
        pltpu.VMEM_SHARED((N, D), jnp.float32),
        pltpu.SemaphoreType.DMA,
        pltpu.SemaphoreType.DMA,
        pltpu.SemaphoreType.DMA,
        pltpu.SemaphoreType.DMA,
        pltpu.SemaphoreType.DMA,
    ],
    compiler_params=_sc_params,
)
def _layer_kernel(row_hbm, col_hbm, ew_hbm, h_hbm, zeros_hbm, outp_hbm,
                  rowidx, colidx, ewbuf, rows, acc,
                  gsem0, gsem1, ssem0, ssem1, isem):
    c = lax.axis_index("c")
    s = lax.axis_index("s")
    wid = s * NC + c
    gsems = (gsem0, gsem1)
    ssems = (ssem0, ssem1)

    pltpu.sync_copy(zeros_hbm.at[pl.ds(s * ROWS_PER_TILE, ROWS_PER_TILE)],
                    acc.at[pl.ds(s * ROWS_PER_TILE, ROWS_PER_TILE)])
    plsc.subcore_barrier()

    lo = wid * GPW

    def fire_chunk(ci, cp):
        gb = lo + ci * CH
        pltpu.async_copy(row_hbm.at[pl.ds(gb, CH)], rowidx.at[cp], isem)
        pltpu.async_copy(col_hbm.at[pl.ds(gb, CH)], colidx.at[cp], isem)
        pltpu.async_copy(ew_hbm.at[pl.ds(gb, CH)], ewbuf.at[cp], isem)

    def wait_chunk(ci, cp):
        gb = lo + ci * CH
        pltpu.make_async_copy(row_hbm.at[pl.ds(gb, CH)], rowidx.at[cp],
                              isem).wait()
        pltpu.make_async_copy(col_hbm.at[pl.ds(gb, CH)], colidx.at[cp],
                              isem).wait()
        pltpu.make_async_copy(ew_hbm.at[pl.ds(gb, CH)], ewbuf.at[cp],
                              isem).wait()

    def fire_gather(cp, o, b):
        pltpu.async_copy(h_hbm.at[rowidx.at[cp, o]], rows.at[b], gsems[b])

    def wait_gather(cp, o, b):
        pltpu.make_async_copy(h_hbm.at[rowidx.at[cp, o]], rows.at[b],
                              gsems[b]).wait()

    def fire_scatter(cp, o, b):
        pltpu.async_copy(rows.at[b], acc.at[colidx.at[cp, o]], ssems[b],
                         add=True)

    def wait_scatter(cp, o, b):
        pltpu.make_async_copy(rows.at[b], acc.at[colidx.at[cp, o]],
                              ssems[b]).wait()

    def scale(cp, o, b):
        def sbody(k, carry2):
            e0 = k * 4
            for j in range(4):
                w16 = plsc.load_gather(
                    ewbuf.at[cp, o], [jnp.full((16,), e0 + j, jnp.int32)])
                for f in range(D // 16):
                    sl = pl.ds(f * 16, 16)
                    rows[b, e0 + j, sl] = rows[b, e0 + j, sl] * w16
            return carry2

        lax.fori_loop(0, G // 4, sbody, 0)

    def cbody(ci, carry):
        fire_chunk(ci, 0)
        wait_chunk(ci, 0)
        fire_gather(0, 0, 0)
        for o in range(CH):
            b = o % 2
            nb = 1 - b
            if o + 1 < CH:
                if o >= 1:
                    wait_scatter(0, o - 1, nb)
                fire_gather(0, o + 1, nb)
            wait_gather(0, o, b)
            scale(0, o, b)
            fire_scatter(0, o, b)
        wait_scatter(0, CH - 2, 0)
        wait_scatter(0, CH - 1, 1)
        return carry

    lax.fori_loop(0, NCHUNK, cbody, 0)
    plsc.subcore_barrier()

    pltpu.sync_copy(acc.at[pl.ds(s * ROWS_PER_TILE, ROWS_PER_TILE)],
                    outp_hbm.at[c, pl.ds(s * ROWS_PER_TILE, ROWS_PER_TILE)])


def _dis_body(degp_ref, dis_ref):
    deg = jnp.sum(degp_ref[...], axis=0, keepdims=True) + 1.0
    safe = jnp.where(deg > 0, deg, 1.0)
    dis_ref[...] = jnp.where(deg > 0, lax.rsqrt(safe), 0.0)


def _dis_call(degp):
    return pl.pallas_call(
        _dis_body,
        out_shape=jax.ShapeDtypeStruct((1, N), jnp.float32),
    )(degp)


_BLK = 2000
_NBLK = N // _BLK


def _mm_body(x_ref, w_ref, disc_ref, out_ref):
    h = lax.dot_general(x_ref[...], w_ref[...], (((1,), (1,)), ((), ())),
                        preferred_element_type=jnp.float32)
    out_ref[...] = h * disc_ref[...]


def _mm_call(x, w, disc):
    return pl.pallas_call(
        _mm_body,
        grid=(_NBLK,),
        in_specs=[
            pl.BlockSpec((_BLK, D), lambda i: (i, 0)),
            pl.BlockSpec((D, D), lambda i: (0, 0)),
            pl.BlockSpec((_BLK, 1), lambda i: (i, 0)),
        ],
        out_specs=pl.BlockSpec((_BLK, D), lambda i: (i, 0)),
        out_shape=jax.ShapeDtypeStruct((N, D), jnp.float32),
    )(x, w, disc)


def _mid_body(p_ref, hp_ref, disc_ref, b_ref, w_ref, out_ref):
    srow = p_ref[0] + p_ref[1] + hp_ref[...]
    z = jnp.maximum(disc_ref[...] * srow + b_ref[...], 0.0)
    h2 = lax.dot_general(z, w_ref[...], (((1,), (1,)), ((), ())),
                         preferred_element_type=jnp.float32)
    out_ref[...] = h2 * disc_ref[...]


def _mid_call(p, hp, disc, b, w):
    return pl.pallas_call(
        _mid_body,
        grid=(_NBLK,),
        in_specs=[
            pl.BlockSpec((NC, _BLK, D), lambda i: (0, i, 0)),
            pl.BlockSpec((_BLK, D), lambda i: (i, 0)),
            pl.BlockSpec((_BLK, 1), lambda i: (i, 0)),
            pl.BlockSpec((1, D), lambda i: (0, 0)),
            pl.BlockSpec((D, D), lambda i: (0, 0)),
        ],
        out_specs=pl.BlockSpec((_BLK, D), lambda i: (i, 0)),
        out_shape=jax.ShapeDtypeStruct((N, D), jnp.float32),
    )(p, hp, disc, b, w)


def _final_body(p_ref, hp_ref, disc_ref, b_ref, out_ref):
    srow = p_ref[0] + p_ref[1] + hp_ref[...]
    out_ref[...] = jnp.maximum(disc_ref[...] * srow + b_ref[...], 0.0)


def _final_call(p, hp, disc, b):
    return pl.pallas_call(
        _final_body,
        grid=(_NBLK,),
        in_specs=[
            pl.BlockSpec((NC, _BLK, D), lambda i: (0, i, 0)),
            pl.BlockSpec((_BLK, D), lambda i: (i, 0)),
            pl.BlockSpec((_BLK, 1), lambda i: (i, 0)),
            pl.BlockSpec((1, D), lambda i: (0, 0)),
        ],
        out_specs=pl.BlockSpec((_BLK, D), lambda i: (i, 0)),
        out_shape=jax.ShapeDtypeStruct((N, D), jnp.float32),
    )(p, hp, disc, b)


def kernel(x, edge_index, edge_weights, W1, b1, W2, b2):
    npad = EPAD - E
    spread = (jnp.arange(npad, dtype=jnp.int32) * 16) % N
    row = jnp.concatenate(
        [edge_index[0], spread]).reshape(NGROUPS, G)
    col = jnp.concatenate(
        [edge_index[1], spread]).reshape(NGROUPS, G)
    ew = jnp.concatenate(
        [edge_weights, jnp.zeros((npad,), jnp.float32)]).reshape(NGROUPS, G)
    zeros_nd = jnp.zeros((N, D), jnp.float32)

    degp = _deg_kernel(col, ew)
    dis = _dis_call(degp)
    disc = dis.reshape(N, 1)

    h1p = _mm_call(x, W1, disc)
    p1 = _layer_kernel(row, col, ew, h1p, zeros_nd)
    h2p = _mid_call(p1, h1p, disc, b1.reshape(1, D), W2)
    p2 = _layer_kernel(row, col, ew, h2p, zeros_nd)
    return _final_call(p2, h2p, disc, b2.reshape(1, D))


# speedup vs baseline: 2.7116x; 1.0519x over previous
import functools

import jax
import jax.numpy as jnp
from jax import lax
from jax.experimental import pallas as pl
from jax.experimental.pallas import tpu as pltpu
from jax.experimental.pallas import tpu_sc as plsc

N = 10000
E = 320000
D = 128
NC = 2
NS = 16
NW = NC * NS
G = 128
GPW = 80
NGROUPS = NW * GPW
EPAD = NGROUPS * G
CH = 16
NCHUNK = GPW // CH

ROWS_PER_TILE = N // NS

_mesh = plsc.VectorSubcoreMesh(core_axis_name="c", subcore_axis_name="s")
_sc_params = pltpu.CompilerParams(needs_layout_passes=False,
                                  use_tc_tiling_on_sc=False)


@functools.partial(
    pl.kernel,
    out_type=jax.ShapeDtypeStruct((NW, N), jnp.float32),
    mesh=_mesh,
    scratch_types=[
        pltpu.VMEM((CH, G), jnp.int32),
        pltpu.VMEM((CH, G), jnp.float32),
        pltpu.VMEM((N,), jnp.float32),
    ],
    compiler_params=_sc_params,
)
def _deg_kernel(col_hbm, ew_hbm, degp_hbm, colbuf, ewbuf, deg_local):
    c = lax.axis_index("c")
    s = lax.axis_index("s")
    wid = s * NC + c

    zero16 = jnp.zeros((16,), jnp.float32)

    def zbody(i, carry):
        deg_local[pl.ds(i * 16, 16)] = zero16
        return carry

    lax.fori_loop(0, N // 16, zbody, 0)

    lo = wid * GPW

    def cbody(ci, carry):
        gb = lo + ci * CH
        pltpu.sync_copy(col_hbm.at[pl.ds(gb, CH)], colbuf)
        pltpu.sync_copy(ew_hbm.at[pl.ds(gb, CH)], ewbuf)

        def inner(k, carry2):
            o = k // (G // 16)
            e0 = (k % (G // 16)) * 16
            cv = colbuf[o, pl.ds(e0, 16)]
            wv = ewbuf[o, pl.ds(e0, 16)]
            plsc.addupdate_scatter(deg_local, [cv], wv)
            return carry2

        lax.fori_loop(0, CH * (G // 16), inner, 0)
        return carry

    lax.fori_loop(0, NCHUNK, cbody, 0)
    pltpu.sync_copy(deg_local, degp_hbm.at[wid])


@functools.partial(
    pl.kernel,
    out_type=jax.ShapeDtypeStruct((NC, N, D), jnp.float32),
    mesh=_mesh,
    scratch_types=[
        pltpu.VMEM((2, CH, G), jnp.int32),
        pltpu.VMEM((2, CH, G), jnp.int32),
        pltpu.VMEM((2, CH, G), jnp.float32),
        pltpu.VMEM((2, G, D), jnp.float32),
        pltpu.VMEM_SHARED((N, D), jnp.float32),
        pltpu.SemaphoreType.DMA,
        pltpu.SemaphoreType.DMA,
        pltpu.SemaphoreType.DMA,
        pltpu.SemaphoreType.DMA,
        pltpu.SemaphoreType.DMA,
    ],
    compiler_params=_sc_params,
)
def _layer_kernel(row_hbm, col_hbm, ew_hbm, h_hbm, zeros_hbm, outp_hbm,
                  rowidx, colidx, ewbuf, rows, acc,
                  gsem0, gsem1, ssem0, ssem1, isem):
    c = lax.axis_index("c")
    s = lax.axis_index("s")
    wid = s * NC + c
    gsems = (gsem0, gsem1)
    ssems = (ssem0, ssem1)

    pltpu.sync_copy(zeros_hbm.at[pl.ds(s * ROWS_PER_TILE, ROWS_PER_TILE)],
                    acc.at[pl.ds(s * ROWS_PER_TILE, ROWS_PER_TILE)])
    plsc.subcore_barrier()

    lo = wid * GPW

    def fire_chunk(ci, cp):
        gb = lo + ci * CH
        pltpu.async_copy(row_hbm.at[pl.ds(gb, CH)], rowidx.at[cp], isem)
        pltpu.async_copy(col_hbm.at[pl.ds(gb, CH)], colidx.at[cp], isem)
        pltpu.async_copy(ew_hbm.at[pl.ds(gb, CH)], ewbuf.at[cp], isem)

    def wait_chunk(ci, cp):
        gb = lo + ci * CH
        pltpu.make_async_copy(row_hbm.at[pl.ds(gb, CH)], rowidx.at[cp],
                              isem).wait()
        pltpu.make_async_copy(col_hbm.at[pl.ds(gb, CH)], colidx.at[cp],
                              isem).wait()
        pltpu.make_async_copy(ew_hbm.at[pl.ds(gb, CH)], ewbuf.at[cp],
                              isem).wait()

    def fire_gather(cp, o, b):
        pltpu.async_copy(h_hbm.at[rowidx.at[cp, o]], rows.at[b], gsems[b])

    def wait_gather(cp, o, b):
        pltpu.make_async_copy(h_hbm.at[rowidx.at[cp, o]], rows.at[b],
                              gsems[b]).wait()

    def fire_scatter(cp, o, b):
        pltpu.async_copy(rows.at[b], acc.at[colidx.at[cp, o]], ssems[b],
                         add=True)

    def wait_scatter(cp, o, b):
        pltpu.make_async_copy(rows.at[b], acc.at[colidx.at[cp, o]],
                              ssems[b]).wait()

    def scale(cp, o, b):
        def sbody(k, carry2):
            e0 = k * 4
            for j in range(4):
                w16 = plsc.load_gather(
                    ewbuf.at[cp, o], [jnp.full((16,), e0 + j, jnp.int32)])
                for f in range(D // 16):
                    sl = pl.ds(f * 16, 16)
                    rows[b, e0 + j, sl] = rows[b, e0 + j, sl] * w16
            return carry2

        lax.fori_loop(0, G // 4, sbody, 0)

    def cbody(ci, carry):
        fire_chunk(ci, 0)
        wait_chunk(ci, 0)
        fire_gather(0, 0, 0)
        for o in range(CH):
            b = o % 2
            nb = 1 - b
            if o + 1 < CH:
                if o >= 1:
                    wait_scatter(0, o - 1, nb)
                fire_gather(0, o + 1, nb)
            wait_gather(0, o, b)
            scale(0, o, b)
            fire_scatter(0, o, b)
        wait_scatter(0, CH - 2, 0)
        wait_scatter(0, CH - 1, 1)
        return carry

    lax.fori_loop(0, NCHUNK, cbody, 0)
    plsc.subcore_barrier()

    pltpu.sync_copy(acc.at[pl.ds(s * ROWS_PER_TILE, ROWS_PER_TILE)],
                    outp_hbm.at[c, pl.ds(s * ROWS_PER_TILE, ROWS_PER_TILE)])


def _dis_body(degp_ref, dis_ref):
    deg = jnp.sum(degp_ref[...], axis=0, keepdims=True) + 1.0
    safe = jnp.where(deg > 0, deg, 1.0)
    dis_ref[...] = jnp.where(deg > 0, lax.rsqrt(safe), 0.0)


def _dis_call(degp):
    return pl.pallas_call(
        _dis_body,
        out_shape=jax.ShapeDtypeStruct((1, N), jnp.float32),
    )(degp)


_BLK = 2000
_NBLK = N // _BLK


def _mm_body(x_ref, w_ref, disc_ref, out_ref):
    h = lax.dot_general(x_ref[...], w_ref[...], (((1,), (1,)), ((), ())),
                        preferred_element_type=jnp.float32)
    out_ref[...] = h * disc_ref[...]


def _mm_call(x, w, disc):
    return pl.pallas_call(
        _mm_body,
        grid=(_NBLK,),
        in_specs=[
            pl.BlockSpec((_BLK, D), lambda i: (i, 0)),
            pl.BlockSpec((D, D), lambda i: (0, 0)),
            pl.BlockSpec((_BLK, 1), lambda i: (i, 0)),
        ],
        out_specs=pl.BlockSpec((_BLK, D), lambda i: (i, 0)),
        out_shape=jax.ShapeDtypeStruct((N, D), jnp.float32),
    )(x, w, disc)


def _mid_body(p_ref, hp_ref, disc_ref, b_ref, w_ref, out_ref):
    srow = p_ref[0] + p_ref[1] + hp_ref[...]
    z = jnp.maximum(disc_ref[...] * srow + b_ref[...], 0.0)
    h2 = lax.dot_general(z, w_ref[...], (((1,), (1,)), ((), ())),
                         preferred_element_type=jnp.float32)
    out_ref[...] = h2 * disc_ref[...]


def _mid_call(p, hp, disc, b, w):
    return pl.pallas_call(
        _mid_body,
        grid=(_NBLK,),
        in_specs=[
            pl.BlockSpec((NC, _BLK, D), lambda i: (0, i, 0)),
            pl.BlockSpec((_BLK, D), lambda i: (i, 0)),
            pl.BlockSpec((_BLK, 1), lambda i: (i, 0)),
            pl.BlockSpec((1, D), lambda i: (0, 0)),
            pl.BlockSpec((D, D), lambda i: (0, 0)),
        ],
        out_specs=pl.BlockSpec((_BLK, D), lambda i: (i, 0)),
        out_shape=jax.ShapeDtypeStruct((N, D), jnp.float32),
    )(p, hp, disc, b, w)


def _final_body(p_ref, hp_ref, disc_ref, b_ref, out_ref):
    srow = p_ref[0] + p_ref[1] + hp_ref[...]
    out_ref[...] = jnp.maximum(disc_ref[...] * srow + b_ref[...], 0.0)


def _final_call(p, hp, disc, b):
    return pl.pallas_call(
        _final_body,
        grid=(_NBLK,),
        in_specs=[
            pl.BlockSpec((NC, _BLK, D), lambda i: (0, i, 0)),
            pl.BlockSpec((_BLK, D), lambda i: (i, 0)),
            pl.BlockSpec((_BLK, 1), lambda i: (i, 0)),
            pl.BlockSpec((1, D), lambda i: (0, 0)),
        ],
        out_specs=pl.BlockSpec((_BLK, D), lambda i: (i, 0)),
        out_shape=jax.ShapeDtypeStruct((N, D), jnp.float32),
    )(p, hp, disc, b)


def kernel(x, edge_index, edge_weights, W1, b1, W2, b2):
    npad = EPAD - E
    spread = (jnp.arange(npad, dtype=jnp.int32) * 16) % N
    row = jnp.concatenate(
        [edge_index[0], spread]).reshape(NGROUPS, G)
    col = jnp.concatenate(
        [edge_index[1], spread]).reshape(NGROUPS, G)
    ew = jnp.concatenate(
        [edge_weights, jnp.zeros((npad,), jnp.float32)]).reshape(NGROUPS, G)
    zeros_nd = jnp.zeros((N, D), jnp.float32)

    degp = _deg_kernel(col, ew)
    dis = _dis_call(degp)
    disc = dis.reshape(N, 1)

    h1p = _mm_call(x, W1, disc)
    p1 = _layer_kernel(row, col, ew, h1p, zeros_nd)
    h2p = _mid_call(p1, h1p, disc, b1.reshape(1, D), W2)
    p2 = _layer_kernel(row, col, ew, h2p, zeros_nd)
    return _final_call(p2, h2p, disc, b2.reshape(1, D))
